# baseline (speedup 1.0000x reference)
import functools

import numpy as np
import jax

_LUT_NP = np.minimum(
    (np.arange(128)[:, None]
     >= np.array([1, 2, 3, 4, 8, 16, 32, 64])[None, :]).sum(1), 8
).astype(np.int32)
import jax.numpy as jnp
from jax import lax
from jax.experimental import pallas as pl
from jax.experimental.pallas import tpu as pltpu
from jax.experimental.pallas import tpu_sc as plsc

_NC, _NS, _L = 2, 16, 16


def kernel(lengths, table):
    n = lengths.shape[0]
    rows, d = table.shape
    nw = _NC * _NS
    n_per_w = n // nw
    groups = n_per_w // _L

    flat = np.arange(_L * d)
    rk_np = (flat // d).reshape(d, _L).astype(np.int32)
    ck_np = (flat % d).reshape(d, _L).astype(np.int32)

    mesh = plsc.VectorSubcoreMesh(
        core_axis_name="c", subcore_axis_name="s",
        num_cores=_NC, num_subcores=_NS)

    @functools.partial(
        pl.kernel,
        out_type=jax.ShapeDtypeStruct((n * d,), jnp.float32),
        mesh=mesh,
        compiler_params=pltpu.CompilerParams(needs_layout_passes=False),
        scratch_types=[
            pltpu.VMEM((n_per_w,), jnp.int32),
            pltpu.VMEM((rows * d,), jnp.float32),
            pltpu.VMEM((n_per_w * d,), jnp.float32),
            pltpu.VMEM((128,), jnp.int32),
        ],
    )
    def run(lengths_hbm, table_hbm, lut_hbm, out_hbm, len_v, tab_v, out_v,
            lut_v):
        wid = lax.axis_index("s") * _NC + lax.axis_index("c")
        base = wid * n_per_w
        pltpu.sync_copy(lengths_hbm.at[pl.ds(base, n_per_w)], len_v)
        pltpu.sync_copy(table_hbm, tab_v)
        pltpu.sync_copy(lut_hbm, lut_v)

        lane = lax.iota(jnp.int32, _L)
        rk_c, fpos_c = [], []
        for k in range(d):
            p = lane + (_L * k)
            rk = (p * 13108) >> 18
            rk_c.append(rk)
            fpos_c.append(p - rk * d)

        @plsc.parallel_loop(0, groups, 1, unroll=1)
        def body(g):
            lv = len_v[pl.ds(g * _L, _L)]
            tpos = plsc.load_gather(lut_v, [lv])
            gbase = g * (_L * d)
            vals = []
            for k in range(d):
                fpos = tpos.at[rk_c[k]].get(mode="promise_in_bounds") + fpos_c[k]
                vals.append(plsc.load_gather(tab_v, [fpos]))
            for k in range(d):
                out_v[pl.ds(gbase + k * _L, _L)] = vals[k]

        pltpu.sync_copy(out_v, out_hbm.at[pl.ds(base * d, n_per_w * d)])

    lut = jnp.asarray(_LUT_NP * d)
    return run(lengths, table.reshape(-1), lut).reshape(n, d)


# speedup vs baseline: 1.0249x; 1.0249x over previous
import functools

import jax
import jax.numpy as jnp
from jax import lax
from jax.experimental import pallas as pl
from jax.experimental.pallas import tpu as pltpu
from jax.experimental.pallas import tpu_sc as plsc

_NC = 2
_NS = 16
_L = 16


def kernel(lengths, table):
    n = lengths.shape[0]
    rows, d = table.shape
    nw = _NC * _NS
    n_per_w = n // nw
    groups = n_per_w // _L

    mesh = plsc.VectorSubcoreMesh(
        core_axis_name="c", subcore_axis_name="s",
        num_cores=_NC, num_subcores=_NS)

    @functools.partial(
        pl.kernel,
        out_type=jax.ShapeDtypeStruct((n * d,), jnp.float32),
        mesh=mesh,
        compiler_params=pltpu.CompilerParams(needs_layout_passes=False),
        scratch_types=[
            pltpu.VMEM((n_per_w,), jnp.int32),
            pltpu.VMEM((rows * d,), jnp.float32),
            pltpu.VMEM((n_per_w * d,), jnp.float32),
        ],
    )
    def run(lengths_hbm, table_hbm, out_hbm, len_v, tab_v, out_v):
        wid = lax.axis_index("s") * _NC + lax.axis_index("c")
        base = wid * n_per_w
        pltpu.sync_copy(lengths_hbm.at[pl.ds(base, n_per_w)], len_v)
        pltpu.sync_copy(table_hbm, tab_v)

        lane = lax.iota(jnp.int32, _L)
        rk_c, ck_c = [], []
        for k in range(d):
            p = lane + (_L * k)
            rk = (p * 13108) >> 18
            rk_c.append(rk)
            ck_c.append(p - rk * d)

        @plsc.parallel_loop(0, groups, 1, unroll=1)
        def body(g):
            lv = len_v[pl.ds(g * _L, _L)]
            f = lv.astype(jnp.float32)
            e2 = (lax.bitcast_convert_type(f, jnp.int32) >> 23) - 125
            idx = jnp.where(lv < 4, lv, e2)
            tpos = idx * d
            gbase = g * (_L * d)
            vals = []
            for k in range(d):
                fpos = tpos.at[rk_c[k]].get(mode="promise_in_bounds") + ck_c[k]
                vals.append(plsc.load_gather(tab_v, [fpos]))
            for k in range(d):
                out_v[pl.ds(gbase + k * _L, _L)] = vals[k]

        pltpu.sync_copy(out_v, out_hbm.at[pl.ds(base * d, n_per_w * d)])

    return run(lengths, table.reshape(-1)).reshape(n, d)
